# baseline (speedup 1.0000x reference)
import jax
import jax.numpy as jnp
from jax.experimental import pallas as pl

_BLOCK_S = 128


def _add_kernel(x_ref, pos_ref, o_ref):
    o_ref[...] = x_ref[...] + pos_ref[...][None]


def kernel(x, pos_table):
    B, S, D = x.shape
    pos = pos_table[:S]
    grid = (S // _BLOCK_S,)
    return pl.pallas_call(
        _add_kernel,
        grid=grid,
        in_specs=[
            pl.BlockSpec((B, _BLOCK_S, D), lambda s: (0, s, 0)),
            pl.BlockSpec((_BLOCK_S, D), lambda s: (s, 0)),
        ],
        out_specs=pl.BlockSpec((B, _BLOCK_S, D), lambda s: (0, s, 0)),
        out_shape=jax.ShapeDtypeStruct(x.shape, x.dtype),
    )(x, pos)
